# baseline (speedup 1.0000x reference)
import functools

import jax
import jax.numpy as jnp
from jax import lax
from jax.experimental import pallas as pl
from jax.experimental.pallas import tpu as pltpu
from jax.experimental.pallas import tpu_sc as plsc

VOCAB = 1000
BATCH = 4096
BLOCK = 8
N = BATCH * BLOCK
NC = 2
NS = 16
NW = NC * NS
ROWS_PER_W = N // NW
CHUNK = 64
NCHUNK = ROWS_PER_W // CHUNK

_mesh = plsc.VectorSubcoreMesh(core_axis_name="c", subcore_axis_name="s")


@functools.partial(
    pl.kernel,
    mesh=_mesh,
    out_type=jax.ShapeDtypeStruct((N, VOCAB), jnp.float32),
    scratch_types=[
        pltpu.VMEM((NCHUNK, CHUNK), jnp.int32),
        pltpu.VMEM((CHUNK, VOCAB), jnp.float32),
        pltpu.VMEM((CHUNK, VOCAB), jnp.float32),
        pltpu.SemaphoreType.DMA,
        pltpu.SemaphoreType.DMA,
    ],
    compiler_params=pltpu.CompilerParams(use_tc_tiling_on_sc=False),
)
def _gather_kernel(w_hbm, idx_hbm, out_hbm, idx_v, buf0, buf1, sem0, sem1):
    wid = lax.axis_index("s") * NC + lax.axis_index("c")
    base = wid * ROWS_PER_W
    pltpu.sync_copy(idx_hbm.at[wid], idx_v)
    bufs = (buf0, buf1)
    sems = (sem0, sem1)
    copies = [None, None]
    copies[0] = pltpu.async_copy(w_hbm.at[idx_v.at[0]], buf0, sem0)
    for j in range(NCHUNK):
        cur = j % 2
        nxt = (j + 1) % 2
        if j + 1 < NCHUNK:
            copies[nxt] = pltpu.async_copy(
                w_hbm.at[idx_v.at[j + 1]], bufs[nxt], sems[nxt]
            )
        copies[cur].wait()
        pltpu.sync_copy(bufs[cur], out_hbm.at[pl.ds(base + j * CHUNK, CHUNK)])


def kernel(idx, W):
    flat = idx.reshape(NW, NCHUNK, CHUNK).astype(jnp.int32)
    out = _gather_kernel(W, flat)
    return out.reshape(BATCH, BLOCK, VOCAB)


# speedup vs baseline: 1.0015x; 1.0015x over previous
import functools

import jax
import jax.numpy as jnp
from jax import lax
from jax.experimental import pallas as pl
from jax.experimental.pallas import tpu as pltpu
from jax.experimental.pallas import tpu_sc as plsc

VOCAB = 1000
VPAD = 1024
BATCH = 4096
BLOCK = 8
N = BATCH * BLOCK
NC = 2
NS = 16
NW = NC * NS
ROWS_PER_W = N // NW
CHUNK = 32
NCHUNK = ROWS_PER_W // CHUNK
NBUF = 3

_mesh = plsc.VectorSubcoreMesh(core_axis_name="c", subcore_axis_name="s")


@functools.partial(
    pl.kernel,
    mesh=_mesh,
    out_type=jax.ShapeDtypeStruct((N, VOCAB), jnp.float32),
    scratch_types=[
        pltpu.VMEM((NCHUNK, CHUNK), jnp.int32),
        pltpu.VMEM((CHUNK, VPAD), jnp.float32),
        pltpu.VMEM((CHUNK, VPAD), jnp.float32),
        pltpu.VMEM((CHUNK, VPAD), jnp.float32),
        pltpu.SemaphoreType.DMA,
        pltpu.SemaphoreType.DMA,
        pltpu.SemaphoreType.DMA,
        pltpu.SemaphoreType.DMA,
        pltpu.SemaphoreType.DMA,
        pltpu.SemaphoreType.DMA,
    ],
    compiler_params=pltpu.CompilerParams(use_tc_tiling_on_sc=False),
)
def _gather_kernel(
    w_hbm, idx_hbm, out_hbm, idx_v, b0, b1, b2, gs0, gs1, gs2, ss0, ss1, ss2
):
    wid = lax.axis_index("s") * NC + lax.axis_index("c")
    base = wid * ROWS_PER_W
    pltpu.sync_copy(idx_hbm.at[wid], idx_v)
    bufs = (b0, b1, b2)
    gsems = (gs0, gs1, gs2)
    ssems = (ss0, ss1, ss2)

    def gather(j):
        slot = j % NBUF
        return pltpu.async_copy(w_hbm.at[idx_v.at[j]], bufs[slot], gsems[slot])

    def scatter(j):
        slot = j % NBUF
        return pltpu.async_copy(
            bufs[slot].at[:, pl.ds(0, VOCAB)],
            out_hbm.at[pl.ds(base + j * CHUNK, CHUNK)],
            ssems[slot],
        )

    g = [None] * NCHUNK
    s = [None] * NCHUNK
    waited = [False] * NCHUNK
    g[0] = gather(0)
    g[1] = gather(1)
    for j in range(NCHUNK):
        if j + 2 < NCHUNK:
            if j >= 1:
                s[j - 1].wait()
                waited[j - 1] = True
            g[j + 2] = gather(j + 2)
        g[j].wait()
        s[j] = scatter(j)
    for j in range(NCHUNK):
        if not waited[j]:
            s[j].wait()


def kernel(idx, W):
    w_pad = jnp.pad(W, ((0, 0), (0, VPAD - VOCAB)))
    flat = idx.reshape(NW, NCHUNK, CHUNK).astype(jnp.int32)
    out = _gather_kernel(w_pad, flat)
    return out.reshape(BATCH, BLOCK, VOCAB)


# speedup vs baseline: 1.1854x; 1.1836x over previous
import functools

import jax
import jax.numpy as jnp
from jax import lax
from jax.experimental import pallas as pl
from jax.experimental.pallas import tpu as pltpu
from jax.experimental.pallas import tpu_sc as plsc

VOCAB = 1000
VPAD = 1024
BATCH = 4096
BLOCK = 8
N = BATCH * BLOCK
NC = 2
NS = 16
NW = NC * NS
ROWS_PER_W = N // NW
CHUNK = 32
NCHUNK = ROWS_PER_W // CHUNK
NBUF = 3

_mesh = plsc.VectorSubcoreMesh(core_axis_name="c", subcore_axis_name="s")


@functools.partial(
    pl.kernel,
    mesh=_mesh,
    out_type=jax.ShapeDtypeStruct((N, VPAD), jnp.float32),
    scratch_types=[
        pltpu.VMEM((NCHUNK, CHUNK), jnp.int32),
        pltpu.VMEM((CHUNK, VPAD), jnp.float32),
        pltpu.VMEM((CHUNK, VPAD), jnp.float32),
        pltpu.VMEM((CHUNK, VPAD), jnp.float32),
        pltpu.SemaphoreType.DMA,
        pltpu.SemaphoreType.DMA,
        pltpu.SemaphoreType.DMA,
        pltpu.SemaphoreType.DMA,
        pltpu.SemaphoreType.DMA,
        pltpu.SemaphoreType.DMA,
    ],
)
def _gather_kernel(
    w_hbm, idx_hbm, out_hbm, idx_v, b0, b1, b2, gs0, gs1, gs2, ss0, ss1, ss2
):
    wid = lax.axis_index("s") * NC + lax.axis_index("c")
    base = wid * ROWS_PER_W
    pltpu.sync_copy(idx_hbm.at[wid], idx_v)
    bufs = (b0, b1, b2)
    gsems = (gs0, gs1, gs2)
    ssems = (ss0, ss1, ss2)

    def gather(j):
        slot = j % NBUF
        return pltpu.async_copy(w_hbm.at[idx_v.at[j]], bufs[slot], gsems[slot])

    def scatter(j):
        slot = j % NBUF
        return pltpu.async_copy(
            bufs[slot],
            out_hbm.at[pl.ds(base + j * CHUNK, CHUNK)],
            ssems[slot],
        )

    g = [None] * NCHUNK
    s = [None] * NCHUNK
    waited = [False] * NCHUNK
    g[0] = gather(0)
    g[1] = gather(1)
    for j in range(NCHUNK):
        if j + 2 < NCHUNK:
            if j >= 1:
                s[j - 1].wait()
                waited[j - 1] = True
            g[j + 2] = gather(j + 2)
        g[j].wait()
        s[j] = scatter(j)
    for j in range(NCHUNK):
        if not waited[j]:
            s[j].wait()


def kernel(idx, W):
    w_pad = jnp.pad(W, ((0, 0), (0, VPAD - VOCAB)))
    flat = idx.reshape(NW, NCHUNK, CHUNK).astype(jnp.int32)
    out = _gather_kernel(w_pad, flat)
    return out[:, :VOCAB].reshape(BATCH, BLOCK, VOCAB)
